# baseline (speedup 1.0000x reference)
import functools

import jax
import jax.numpy as jnp
import numpy as np
from jax import lax
from jax.experimental import pallas as pl
from jax.experimental.pallas import tpu as pltpu
from jax.experimental.pallas import tpu_sc as plsc

_TUNNEL_PROB = 1.0 / 137.035999139


def _np_threefry_uniform(seed, n):
    m = np.uint64(0xFFFFFFFF)

    def rotl(x, d):
        return ((x << np.uint64(d)) | (x >> np.uint64(32 - d))) & m

    k0 = np.uint64(np.uint64(seed) >> np.uint64(32))
    k1 = np.uint64(np.uint64(seed) & m)
    ks2 = k0 ^ k1 ^ np.uint64(0x1BD11BDA)
    c64 = np.arange(n, dtype=np.uint64)
    x0 = (c64 >> np.uint64(32)) + k0 & m
    x1 = (c64 & m) + k1 & m
    keys = [(k1, ks2), (ks2, k0), (k0, k1), (k1, ks2), (ks2, k0)]
    rots = ([13, 15, 26, 6], [17, 29, 16, 24])
    for i in range(5):
        for r in rots[i % 2]:
            x0 = (x0 + x1) & m
            x1 = rotl(x1, r) ^ x0
        ka, kb = keys[i]
        x0 = (x0 + ka) & m
        x1 = (x1 + kb + np.uint64(i + 1)) & m
    bits = (x0 ^ x1).astype(np.uint32)
    fb = (bits >> np.uint32(9)) | np.uint32(0x3F800000)
    return fb.view(np.float32) - np.float32(1.0)


_TUNNEL_MASK = _np_threefry_uniform(1234, 4) < _TUNNEL_PROB

_BS = 2048
_S_SC = 2048
_NC, _NSUB = 2, 16
_TILES = _NC * _NSUB
_CROWS = 8
_LANES = 16


def _sc_partial_body(nrows, d, s_base, x_hbm, out_hbm, buf0, buf1, acc,
                     sem0, sem1):
    wid = lax.axis_index("s") * _NC + lax.axis_index("c")
    tpb = _TILES // 4
    b = wid // tpb
    sub = wid % tpb
    row0 = s_base + sub * nrows
    nd = d // _LANES
    nchunks = nrows // _CROWS

    for dl in range(nd):
        acc[pl.ds(dl * _LANES, _LANES)] = jnp.zeros((_LANES,), jnp.float32)

    def _start(c, buf, sem):
        pltpu.make_async_copy(
            x_hbm.at[b, pl.ds(row0 + c * _CROWS, _CROWS), :], buf, sem
        ).start()

    def _wait(buf, sem):
        pltpu.make_async_copy(
            x_hbm.at[b, pl.ds(row0, _CROWS), :], buf, sem).wait()

    def _process(buf):
        for dl in range(nd):
            sl = pl.ds(dl * _LANES, _LANES)
            a = acc[sl]
            for r in range(_CROWS):
                a = a + buf[r, sl]
            acc[sl] = a

    _start(0, buf0, sem0)
    _start(1, buf1, sem1)

    @pl.loop(0, nchunks // 2)
    def _(i):
        c0 = 2 * i
        _wait(buf0, sem0)
        _process(buf0)

        @pl.when(c0 + 2 < nchunks)
        def _():
            _start(c0 + 2, buf0, sem0)

        _wait(buf1, sem1)
        _process(buf1)

        @pl.when(c0 + 3 < nchunks)
        def _():
            _start(c0 + 3, buf1, sem1)

    pltpu.sync_copy(acc, out_hbm.at[wid])


def _sc_partial(x, s_base):
    bsz, _, d = x.shape
    nrows = (_S_SC * bsz) // _TILES
    mesh = plsc.VectorSubcoreMesh(core_axis_name="c", subcore_axis_name="s")
    return pl.kernel(
        functools.partial(_sc_partial_body, nrows, d, s_base),
        out_type=jax.ShapeDtypeStruct((_TILES, d), jnp.float32),
        mesh=mesh,
        scratch_types=[
            pltpu.VMEM((_CROWS, d), jnp.float32),
            pltpu.VMEM((_CROWS, d), jnp.float32),
            pltpu.VMEM((d,), jnp.float32),
            pltpu.SemaphoreType.DMA,
            pltpu.SemaphoreType.DMA,
        ],
    )(x)


def _tc_reduce_body(x_ref, sums_ref, acc_ref):
    j = pl.program_id(1)

    @pl.when(j == 0)
    def _():
        acc_ref[...] = jnp.zeros_like(acc_ref)

    acc_ref[...] += jnp.sum(x_ref[0], axis=0, keepdims=True)

    @pl.when(j == pl.num_programs(1) - 1)
    def _():
        sums_ref[0] = acc_ref[...]


def _tc_reduce(x, s_tc):
    bsz, _, d = x.shape
    bs = next(c for c in (2048, 1024, 512, 256) if s_tc % c == 0)
    ns = s_tc // bs
    return pl.pallas_call(
        _tc_reduce_body,
        grid=(bsz, ns),
        in_specs=[pl.BlockSpec((1, bs, d), lambda b, j: (b, j, 0))],
        out_specs=pl.BlockSpec((1, 1, d), lambda b, j: (b, 0, 0)),
        out_shape=jax.ShapeDtypeStruct((bsz, 1, d), jnp.float32),
        scratch_shapes=[pltpu.VMEM((1, d), jnp.float32)],
        compiler_params=pltpu.CompilerParams(
            dimension_semantics=("arbitrary", "arbitrary")),
    )(x)


def _finalize_body(seq, sums_ref, part_ref, c_ref, scores_ref, idx_ref):
    bsz = sums_ref.shape[0]
    e = c_ref.shape[0]
    tpb = _TILES // bsz
    rows = []
    for b in range(bsz):
        p = jnp.sum(part_ref[b * tpb:(b + 1) * tpb, :], axis=0, keepdims=True)
        rows.append(sums_ref[b] + p)
    total = jnp.concatenate(rows, axis=0)
    q = total * (1.0 / seq)
    qn = q / jnp.maximum(
        jnp.sqrt(jnp.sum(q * q, axis=-1, keepdims=True)), 1e-12)
    c = c_ref[...]
    cn = c / jnp.maximum(
        jnp.sqrt(jnp.sum(c * c, axis=-1, keepdims=True)), 1e-12)
    scores = lax.dot_general(
        qn, cn, (((1,), (1,)), ((), ())),
        preferred_element_type=jnp.float32)
    idx = lax.broadcasted_iota(jnp.int32, (bsz, e), 1)
    max1 = jnp.max(scores, axis=1, keepdims=True)
    i1 = jnp.min(jnp.where(scores == max1, idx, e), axis=1, keepdims=True)
    masked = jnp.where(idx == i1, -jnp.inf, scores)
    max2 = jnp.max(masked, axis=1, keepdims=True)
    i2 = jnp.min(jnp.where(masked == max2, idx, e), axis=1, keepdims=True)
    minv = jnp.min(scores, axis=1, keepdims=True)
    imin = jnp.min(jnp.where(scores == minv, idx, e), axis=1, keepdims=True)
    rows_i = lax.broadcasted_iota(jnp.int32, (bsz, 1), 0)
    tunnel = jnp.zeros((bsz, 1), jnp.bool_)
    for k, m in enumerate(_TUNNEL_MASK.tolist()):
        if m:
            tunnel = jnp.logical_or(tunnel, rows_i == k)
    top0 = jnp.where(tunnel, imin, i1)
    scores_ref[...] = jnp.where((idx == 0) & tunnel, minv, scores)
    idx_ref[...] = jnp.concatenate([top0, i2], axis=1).astype(jnp.int32)


def _finalize(sums, partials, centroids, seq):
    bsz = sums.shape[0]
    e = centroids.shape[0]
    return pl.pallas_call(
        functools.partial(_finalize_body, seq),
        out_shape=[
            jax.ShapeDtypeStruct((bsz, e), jnp.float32),
            jax.ShapeDtypeStruct((bsz, 2), jnp.int32),
        ],
    )(sums, partials, centroids)


def kernel(x, centroids):
    _, seq, _ = x.shape
    s_tc = seq - _S_SC
    partials = _sc_partial(x, s_tc)
    sums = _tc_reduce(x, s_tc)
    scores_t, top_idx = _finalize(sums, partials, centroids, seq)
    return (scores_t, top_idx)


# speedup vs baseline: 1.3333x; 1.3333x over previous
import functools

import jax
import jax.numpy as jnp
import numpy as np
from jax import lax
from jax.experimental import pallas as pl
from jax.experimental.pallas import tpu as pltpu
from jax.experimental.pallas import tpu_sc as plsc

_TUNNEL_PROB = 1.0 / 137.035999139


def _np_threefry_uniform(seed, n):
    m = np.uint64(0xFFFFFFFF)

    def rotl(x, d):
        return ((x << np.uint64(d)) | (x >> np.uint64(32 - d))) & m

    k0 = np.uint64(np.uint64(seed) >> np.uint64(32))
    k1 = np.uint64(np.uint64(seed) & m)
    ks2 = k0 ^ k1 ^ np.uint64(0x1BD11BDA)
    c64 = np.arange(n, dtype=np.uint64)
    x0 = (c64 >> np.uint64(32)) + k0 & m
    x1 = (c64 & m) + k1 & m
    keys = [(k1, ks2), (ks2, k0), (k0, k1), (k1, ks2), (ks2, k0)]
    rots = ([13, 15, 26, 6], [17, 29, 16, 24])
    for i in range(5):
        for r in rots[i % 2]:
            x0 = (x0 + x1) & m
            x1 = rotl(x1, r) ^ x0
        ka, kb = keys[i]
        x0 = (x0 + ka) & m
        x1 = (x1 + kb + np.uint64(i + 1)) & m
    bits = (x0 ^ x1).astype(np.uint32)
    fb = (bits >> np.uint32(9)) | np.uint32(0x3F800000)
    return fb.view(np.float32) - np.float32(1.0)


_TUNNEL_MASK = _np_threefry_uniform(1234, 4) < _TUNNEL_PROB

_BS = 2048
_S_SC = 1024
_NC, _NSUB = 2, 16
_TILES = _NC * _NSUB
_CROWS = 8
_LANES = 16


def _sc_partial_body(nrows, d, s_base, x_hbm, out_hbm, buf0, buf1, acc,
                     sem0, sem1):
    wid = lax.axis_index("s") * _NC + lax.axis_index("c")
    tpb = _TILES // 4
    b = wid // tpb
    sub = wid % tpb
    row0 = s_base + sub * nrows
    nd = d // _LANES
    nchunks = nrows // _CROWS

    for dl in range(nd):
        acc[pl.ds(dl * _LANES, _LANES)] = jnp.zeros((_LANES,), jnp.float32)

    def _start(c, buf, sem):
        pltpu.make_async_copy(
            x_hbm.at[b, pl.ds(row0 + c * _CROWS, _CROWS), :], buf, sem
        ).start()

    def _wait(buf, sem):
        pltpu.make_async_copy(
            x_hbm.at[b, pl.ds(row0, _CROWS), :], buf, sem).wait()

    def _process(buf):
        for dl in range(nd):
            sl = pl.ds(dl * _LANES, _LANES)
            vals = [buf[r, sl] for r in range(_CROWS)]
            while len(vals) > 1:
                vals = [vals[i] + vals[i + 1] for i in range(0, len(vals), 2)]
            acc[sl] = acc[sl] + vals[0]

    _start(0, buf0, sem0)
    _start(1, buf1, sem1)

    @pl.loop(0, nchunks // 2)
    def _(i):
        c0 = 2 * i
        _wait(buf0, sem0)
        _process(buf0)

        @pl.when(c0 + 2 < nchunks)
        def _():
            _start(c0 + 2, buf0, sem0)

        _wait(buf1, sem1)
        _process(buf1)

        @pl.when(c0 + 3 < nchunks)
        def _():
            _start(c0 + 3, buf1, sem1)

    pltpu.sync_copy(acc, out_hbm.at[wid])


def _sc_partial(x, s_base):
    bsz, _, d = x.shape
    nrows = (_S_SC * bsz) // _TILES
    mesh = plsc.VectorSubcoreMesh(core_axis_name="c", subcore_axis_name="s")
    return pl.kernel(
        functools.partial(_sc_partial_body, nrows, d, s_base),
        out_type=jax.ShapeDtypeStruct((_TILES, d), jnp.float32),
        mesh=mesh,
        scratch_types=[
            pltpu.VMEM((_CROWS, d), jnp.float32),
            pltpu.VMEM((_CROWS, d), jnp.float32),
            pltpu.VMEM((d,), jnp.float32),
            pltpu.SemaphoreType.DMA,
            pltpu.SemaphoreType.DMA,
        ],
    )(x)


def _tc_reduce_body(x_ref, sums_ref, acc_ref):
    j = pl.program_id(1)

    @pl.when(j == 0)
    def _():
        acc_ref[...] = jnp.zeros_like(acc_ref)

    acc_ref[...] += jnp.sum(x_ref[0], axis=0, keepdims=True)

    @pl.when(j == pl.num_programs(1) - 1)
    def _():
        sums_ref[0] = acc_ref[...]


def _tc_reduce(x, s_tc):
    bsz, _, d = x.shape
    bs = next(c for c in (2048, 1024, 512, 256) if s_tc % c == 0)
    ns = s_tc // bs
    return pl.pallas_call(
        _tc_reduce_body,
        grid=(bsz, ns),
        in_specs=[pl.BlockSpec((1, bs, d), lambda b, j: (b, j, 0))],
        out_specs=pl.BlockSpec((1, 1, d), lambda b, j: (b, 0, 0)),
        out_shape=jax.ShapeDtypeStruct((bsz, 1, d), jnp.float32),
        scratch_shapes=[pltpu.VMEM((1, d), jnp.float32)],
        compiler_params=pltpu.CompilerParams(
            dimension_semantics=("arbitrary", "arbitrary")),
    )(x)


def _finalize_body(seq, sums_ref, part_ref, c_ref, scores_ref, idx_ref):
    bsz = sums_ref.shape[0]
    e = c_ref.shape[0]
    tpb = _TILES // bsz
    rows = []
    for b in range(bsz):
        p = jnp.sum(part_ref[b * tpb:(b + 1) * tpb, :], axis=0, keepdims=True)
        rows.append(sums_ref[b] + p)
    total = jnp.concatenate(rows, axis=0)
    q = total * (1.0 / seq)
    qn = q / jnp.maximum(
        jnp.sqrt(jnp.sum(q * q, axis=-1, keepdims=True)), 1e-12)
    c = c_ref[...]
    cn = c / jnp.maximum(
        jnp.sqrt(jnp.sum(c * c, axis=-1, keepdims=True)), 1e-12)
    scores = lax.dot_general(
        qn, cn, (((1,), (1,)), ((), ())),
        preferred_element_type=jnp.float32)
    idx = lax.broadcasted_iota(jnp.int32, (bsz, e), 1)
    max1 = jnp.max(scores, axis=1, keepdims=True)
    i1 = jnp.min(jnp.where(scores == max1, idx, e), axis=1, keepdims=True)
    masked = jnp.where(idx == i1, -jnp.inf, scores)
    max2 = jnp.max(masked, axis=1, keepdims=True)
    i2 = jnp.min(jnp.where(masked == max2, idx, e), axis=1, keepdims=True)
    minv = jnp.min(scores, axis=1, keepdims=True)
    imin = jnp.min(jnp.where(scores == minv, idx, e), axis=1, keepdims=True)
    rows_i = lax.broadcasted_iota(jnp.int32, (bsz, 1), 0)
    tunnel = jnp.zeros((bsz, 1), jnp.bool_)
    for k, m in enumerate(_TUNNEL_MASK.tolist()):
        if m:
            tunnel = jnp.logical_or(tunnel, rows_i == k)
    top0 = jnp.where(tunnel, imin, i1)
    scores_ref[...] = jnp.where((idx == 0) & tunnel, minv, scores)
    idx_ref[...] = jnp.concatenate([top0, i2], axis=1).astype(jnp.int32)


def _finalize(sums, partials, centroids, seq):
    bsz = sums.shape[0]
    e = centroids.shape[0]
    return pl.pallas_call(
        functools.partial(_finalize_body, seq),
        out_shape=[
            jax.ShapeDtypeStruct((bsz, e), jnp.float32),
            jax.ShapeDtypeStruct((bsz, 2), jnp.int32),
        ],
    )(sums, partials, centroids)


def kernel(x, centroids):
    _, seq, _ = x.shape
    s_tc = seq - _S_SC
    partials = _sc_partial(x, s_tc)
    sums = _tc_reduce(x, s_tc)
    scores_t, top_idx = _finalize(sums, partials, centroids, seq)
    return (scores_t, top_idx)


# speedup vs baseline: 1.6176x; 1.2133x over previous
import jax
import jax.numpy as jnp
import numpy as np
from jax import lax
from jax.experimental import pallas as pl
from jax.experimental.pallas import tpu as pltpu

_TUNNEL_PROB = 1.0 / 137.035999139


def _np_threefry_uniform(seed, n):
    m = np.uint64(0xFFFFFFFF)

    def rotl(x, d):
        return ((x << np.uint64(d)) | (x >> np.uint64(32 - d))) & m

    k0 = np.uint64(np.uint64(seed) >> np.uint64(32))
    k1 = np.uint64(np.uint64(seed) & m)
    ks2 = k0 ^ k1 ^ np.uint64(0x1BD11BDA)
    c64 = np.arange(n, dtype=np.uint64)
    x0 = (c64 >> np.uint64(32)) + k0 & m
    x1 = (c64 & m) + k1 & m
    keys = [(k1, ks2), (ks2, k0), (k0, k1), (k1, ks2), (ks2, k0)]
    rots = ([13, 15, 26, 6], [17, 29, 16, 24])
    for i in range(5):
        for r in rots[i % 2]:
            x0 = (x0 + x1) & m
            x1 = rotl(x1, r) ^ x0
        ka, kb = keys[i]
        x0 = (x0 + ka) & m
        x1 = (x1 + kb + np.uint64(i + 1)) & m
    bits = (x0 ^ x1).astype(np.uint32)
    fb = (bits >> np.uint32(9)) | np.uint32(0x3F800000)
    return fb.view(np.float32) - np.float32(1.0)


_TUNNEL_MASK = _np_threefry_uniform(1234, 4) < _TUNNEL_PROB

_BS = 2048


def _router_body(x_ref, c_ref, scores_ref, idx_ref, acc_ref, sums_ref):
    b = pl.program_id(0)
    j = pl.program_id(1)
    nb = pl.num_programs(0)
    ns = pl.num_programs(1)

    @pl.when(j == 0)
    def _():
        acc_ref[...] = jnp.zeros_like(acc_ref)

    acc_ref[...] += jnp.sum(x_ref[0], axis=0, keepdims=True)

    @pl.when(j == ns - 1)
    def _():
        sums_ref[pl.ds(b, 1), :] = acc_ref[...]

    @pl.when((b == nb - 1) & (j == ns - 1))
    def _():
        seq = x_ref.shape[1] * ns
        e = c_ref.shape[0]
        q = sums_ref[...] * (1.0 / seq)
        qn = q / jnp.maximum(
            jnp.sqrt(jnp.sum(q * q, axis=-1, keepdims=True)), 1e-12)
        c = c_ref[...]
        cn = c / jnp.maximum(
            jnp.sqrt(jnp.sum(c * c, axis=-1, keepdims=True)), 1e-12)
        scores = lax.dot_general(
            qn, cn, (((1,), (1,)), ((), ())),
            preferred_element_type=jnp.float32)
        idx = lax.broadcasted_iota(jnp.int32, (nb, e), 1)
        max1 = jnp.max(scores, axis=1, keepdims=True)
        i1 = jnp.min(jnp.where(scores == max1, idx, e), axis=1, keepdims=True)
        masked = jnp.where(idx == i1, -jnp.inf, scores)
        max2 = jnp.max(masked, axis=1, keepdims=True)
        i2 = jnp.min(jnp.where(masked == max2, idx, e), axis=1, keepdims=True)
        minv = jnp.min(scores, axis=1, keepdims=True)
        imin = jnp.min(jnp.where(scores == minv, idx, e),
                       axis=1, keepdims=True)
        rows_i = lax.broadcasted_iota(jnp.int32, (nb, 1), 0)
        tunnel = jnp.zeros((nb, 1), jnp.bool_)
        for k, msk in enumerate(_TUNNEL_MASK.tolist()):
            if msk:
                tunnel = jnp.logical_or(tunnel, rows_i == k)
        top0 = jnp.where(tunnel, imin, i1)
        scores_ref[...] = jnp.where((idx == 0) & tunnel, minv, scores)
        idx_ref[...] = jnp.concatenate([top0, i2], axis=1).astype(jnp.int32)


def kernel(x, centroids):
    bsz, seq, d = x.shape
    e = centroids.shape[0]
    ns = seq // _BS
    scores_t, top_idx = pl.pallas_call(
        _router_body,
        grid=(bsz, ns),
        in_specs=[
            pl.BlockSpec((1, _BS, d), lambda b, j: (b, j, 0)),
            pl.BlockSpec((e, d), lambda b, j: (0, 0)),
        ],
        out_specs=[
            pl.BlockSpec((bsz, e), lambda b, j: (0, 0)),
            pl.BlockSpec((bsz, 2), lambda b, j: (0, 0)),
        ],
        out_shape=[
            jax.ShapeDtypeStruct((bsz, e), jnp.float32),
            jax.ShapeDtypeStruct((bsz, 2), jnp.int32),
        ],
        scratch_shapes=[
            pltpu.VMEM((1, d), jnp.float32),
            pltpu.VMEM((bsz, d), jnp.float32),
        ],
        compiler_params=pltpu.CompilerParams(
            dimension_semantics=("arbitrary", "arbitrary")),
    )(x, centroids)
    return (scores_t, top_idx)
